# Initial kernel scaffold; baseline (speedup 1.0000x reference)
import jax
import jax.numpy as jnp
from jax.experimental import pallas as pl


def kernel(x, edge_index, fc0_w, fc0_b, layer_ws, fc1_w, fc1_b):
    raise NotImplementedError("write your pallas kernel here")


# speedup vs baseline: 9.2036x; 9.2036x over previous
import functools

import jax
import jax.numpy as jnp
import numpy as np
from jax import lax
from jax.experimental import pallas as pl
from jax.experimental.pallas import tpu as pltpu
from jax.experimental.pallas import tpu_sc as plsc

N = 10000
D_IN = 128
D_H = 64
N_CLS = 16
NUM_LAYERS = 8
ALPHA = 0.1
LAMBDA = 0.5

NC = 2
NS = 16
NW = NC * NS
CH = 128
NCHUNK = 79
EPT = NCHUNK * CH
ET = NW * EPT
N_PAD = 10240
RPT = N_PAD // NS

_MESH = plsc.VectorSubcoreMesh(core_axis_name="c", subcore_axis_name="s")
_SC_PARAMS = pltpu.CompilerParams(use_tc_tiling_on_sc=False)


@functools.partial(
    pl.kernel,
    out_type=(
        jax.ShapeDtypeStruct((NC, N_PAD), jnp.float32),
        jax.ShapeDtypeStruct((NC, N_PAD), jnp.float32),
    ),
    mesh=_MESH,
    scratch_types=[
        pltpu.VMEM((NCHUNK, CH), jnp.int32),
        pltpu.VMEM((NCHUNK, CH), jnp.int32),
        pltpu.VMEM((CH,), jnp.float32),
        pltpu.VMEM((RPT,), jnp.float32),
        pltpu.VMEM_SHARED((N_PAD,), jnp.float32),
        pltpu.VMEM_SHARED((N_PAD,), jnp.float32),
    ],
    compiler_params=_SC_PARAMS,
)
def _sc_degrees(src_hbm, dst_hbm, dego_hbm, degi_hbm,
                src_v, dst_v, ones_v, zb_v, dego_sh, degi_sh):
    c = lax.axis_index("c")
    s = lax.axis_index("s")
    wid = c * NS + s
    pltpu.sync_copy(src_hbm.at[wid], src_v)
    pltpu.sync_copy(dst_hbm.at[wid], dst_v)
    for j in range(CH // 16):
        ones_v[pl.ds(j * 16, 16)] = jnp.ones((16,), jnp.float32)

    def _zero(i, carry):
        zb_v[pl.ds(i * 16, 16)] = jnp.zeros((16,), jnp.float32)
        return carry

    lax.fori_loop(0, RPT // 16, _zero, 0)
    pltpu.sync_copy(zb_v, dego_sh.at[pl.ds(s * RPT, RPT)])
    pltpu.sync_copy(zb_v, degi_sh.at[pl.ds(s * RPT, RPT)])
    plsc.subcore_barrier()

    def _body(ci, carry):
        pltpu.sync_copy(ones_v, dego_sh.at[src_v.at[ci]], add=True)
        pltpu.sync_copy(ones_v, degi_sh.at[dst_v.at[ci]], add=True)
        return carry

    lax.fori_loop(0, NCHUNK, _body, 0)
    plsc.subcore_barrier()
    pltpu.sync_copy(dego_sh.at[pl.ds(s * RPT, RPT)],
                    dego_hbm.at[c, pl.ds(s * RPT, RPT)])
    pltpu.sync_copy(degi_sh.at[pl.ds(s * RPT, RPT)],
                    degi_hbm.at[c, pl.ds(s * RPT, RPT)])


@functools.partial(
    pl.kernel,
    out_type=jax.ShapeDtypeStruct((NC, N_PAD, D_H), jnp.float32),
    mesh=_MESH,
    scratch_types=[
        pltpu.VMEM((NCHUNK, CH), jnp.int32),
        pltpu.VMEM((NCHUNK, CH), jnp.int32),
        pltpu.VMEM((CH, D_H), jnp.float32),
        pltpu.VMEM_SHARED((N_PAD, D_H), jnp.float32),
    ],
    compiler_params=_SC_PARAMS,
)
def _sc_gather_scatter(g_hbm, src_hbm, dst_hbm, z_hbm, out_hbm,
                       src_v, dst_v, buf_v, agg_sh):
    c = lax.axis_index("c")
    s = lax.axis_index("s")
    wid = c * NS + s
    pltpu.sync_copy(src_hbm.at[wid], src_v)
    pltpu.sync_copy(dst_hbm.at[wid], dst_v)
    pltpu.sync_copy(z_hbm.at[pl.ds(s * RPT, RPT)],
                    agg_sh.at[pl.ds(s * RPT, RPT)])
    plsc.subcore_barrier()

    def _body(ci, carry):
        pltpu.sync_copy(g_hbm.at[src_v.at[ci]], buf_v)
        pltpu.sync_copy(buf_v, agg_sh.at[dst_v.at[ci]], add=True)
        return carry

    lax.fori_loop(0, NCHUNK, _body, 0)
    plsc.subcore_barrier()
    pltpu.sync_copy(agg_sh.at[pl.ds(s * RPT, RPT)],
                    out_hbm.at[c, pl.ds(s * RPT, RPT)])


def _tc_pre_body(x_ref, w_ref, b_ref, go_ref, gi_ref,
                 h0_ref, g_ref, dsrc_ref, ddst_ref):
    h = jnp.dot(x_ref[...], w_ref[...], preferred_element_type=jnp.float32)
    h = jnp.maximum(h + b_ref[...][None, :], 0.0)
    dego = (go_ref[0, :] + go_ref[1, :])[:N]
    degi = (gi_ref[0, :] + gi_ref[1, :])[:N]
    dsrc = lax.rsqrt(jnp.where(dego > 0, dego, 1.0))[:, None]
    ddst = lax.rsqrt(jnp.where(degi > 0, degi, 1.0))[:, None]
    h0_ref[...] = h
    g_ref[...] = jnp.concatenate(
        [h * dsrc, jnp.zeros((N_PAD - N, D_H), jnp.float32)], axis=0)
    dsrc_ref[...] = dsrc
    ddst_ref[...] = ddst


def _tc_pre(x, fc0_w, fc0_b, dego_p, degi_p):
    return pl.pallas_call(
        _tc_pre_body,
        out_shape=(
            jax.ShapeDtypeStruct((N, D_H), jnp.float32),
            jax.ShapeDtypeStruct((N_PAD, D_H), jnp.float32),
            jax.ShapeDtypeStruct((N, 1), jnp.float32),
            jax.ShapeDtypeStruct((N, 1), jnp.float32),
        ),
    )(x, fc0_w, fc0_b, dego_p, degi_p)


def _tc_layer_body(beta, part_ref, h0_ref, dsrc_ref, ddst_ref, w_ref, g_ref):
    agg = (part_ref[0, :N, :] + part_ref[1, :N, :]) * ddst_ref[...]
    feat = (1.0 - ALPHA) * agg + ALPHA * h0_ref[...]
    t = jnp.dot(feat, w_ref[...], preferred_element_type=jnp.float32)
    h = jnp.maximum((1.0 - beta) * feat + beta * t, 0.0)
    g_ref[...] = jnp.concatenate(
        [h * dsrc_ref[...], jnp.zeros((N_PAD - N, D_H), jnp.float32)], axis=0)


def _tc_layer(part, h0, dsrc, ddst, w, beta):
    return pl.pallas_call(
        functools.partial(_tc_layer_body, beta),
        out_shape=jax.ShapeDtypeStruct((N_PAD, D_H), jnp.float32),
    )(part, h0, dsrc, ddst, w)


def _tc_last_body(beta, part_ref, h0_ref, ddst_ref, w_ref,
                  fc1w_ref, fc1b_ref, out_ref):
    agg = (part_ref[0, :N, :] + part_ref[1, :N, :]) * ddst_ref[...]
    feat = (1.0 - ALPHA) * agg + ALPHA * h0_ref[...]
    t = jnp.dot(feat, w_ref[...], preferred_element_type=jnp.float32)
    h = jnp.maximum((1.0 - beta) * feat + beta * t, 0.0)
    o = jnp.dot(h, fc1w_ref[...], preferred_element_type=jnp.float32)
    out_ref[...] = jnp.maximum(o + fc1b_ref[...][None, :], 0.0)


def _tc_last(part, h0, ddst, w, fc1_w, fc1_b, beta):
    return pl.pallas_call(
        functools.partial(_tc_last_body, beta),
        out_shape=jax.ShapeDtypeStruct((N, N_CLS), jnp.float32),
    )(part, h0, ddst, w, fc1_w, fc1_b)


def kernel(x, edge_index, fc0_w, fc0_b, layer_ws, fc1_w, fc1_b):
    e = edge_index.shape[1]
    pad = ET - e
    pad_idx = N + (jnp.arange(pad, dtype=jnp.int32) % (N_PAD - N))
    src_r = jnp.concatenate([edge_index[0], pad_idx]).reshape(NW, NCHUNK, CH)
    dst_r = jnp.concatenate([edge_index[1], pad_idx]).reshape(NW, NCHUNK, CH)
    zeros2d = jnp.zeros((N_PAD, D_H), jnp.float32)

    dego_p, degi_p = _sc_degrees(src_r, dst_r)
    h0, g, dsrc, ddst = _tc_pre(x, fc0_w, fc0_b, dego_p, degi_p)
    for i in range(NUM_LAYERS - 2):
        beta = float(np.log(LAMBDA / (i + 1) + 1.0))
        part = _sc_gather_scatter(g, src_r, dst_r, zeros2d)
        if i < NUM_LAYERS - 3:
            g = _tc_layer(part, h0, dsrc, ddst, layer_ws[i], beta)
        else:
            out = _tc_last(part, h0, ddst, layer_ws[i], fc1_w, fc1_b, beta)
    return out


# speedup vs baseline: 13.3019x; 1.4453x over previous
import functools

import jax
import jax.numpy as jnp
import numpy as np
from jax import lax
from jax.experimental import pallas as pl
from jax.experimental.pallas import tpu as pltpu
from jax.experimental.pallas import tpu_sc as plsc

N = 10000
D_IN = 128
D_H = 64
N_CLS = 16
NUM_LAYERS = 8
ALPHA = 0.1
LAMBDA = 0.5

NC = 2
NS = 16
NW = NC * NS
CH = 128
NCHUNK = 80
EPT = NCHUNK * CH
ET = NW * EPT
N_PAD = 10240
RPT = N_PAD // NS

_MESH = plsc.VectorSubcoreMesh(core_axis_name="c", subcore_axis_name="s")
_SC_PARAMS = pltpu.CompilerParams(use_tc_tiling_on_sc=False)


@functools.partial(
    pl.kernel,
    out_type=(
        jax.ShapeDtypeStruct((NC, N_PAD), jnp.float32),
        jax.ShapeDtypeStruct((NC, N_PAD), jnp.float32),
    ),
    mesh=_MESH,
    scratch_types=[
        pltpu.VMEM((NCHUNK, CH), jnp.int32),
        pltpu.VMEM((NCHUNK, CH), jnp.int32),
        pltpu.VMEM((CH,), jnp.float32),
        pltpu.VMEM((RPT,), jnp.float32),
        pltpu.VMEM_SHARED((N_PAD,), jnp.float32),
        pltpu.VMEM_SHARED((N_PAD,), jnp.float32),
    ],
    compiler_params=_SC_PARAMS,
)
def _sc_degrees(src_hbm, dst_hbm, dego_hbm, degi_hbm,
                src_v, dst_v, ones_v, zb_v, dego_sh, degi_sh):
    c = lax.axis_index("c")
    s = lax.axis_index("s")
    wid = c * NS + s
    pltpu.sync_copy(src_hbm.at[wid], src_v)
    pltpu.sync_copy(dst_hbm.at[wid], dst_v)
    for j in range(CH // 16):
        ones_v[pl.ds(j * 16, 16)] = jnp.ones((16,), jnp.float32)

    def _zero(i, carry):
        zb_v[pl.ds(i * 16, 16)] = jnp.zeros((16,), jnp.float32)
        return carry

    lax.fori_loop(0, RPT // 16, _zero, 0)
    pltpu.sync_copy(zb_v, dego_sh.at[pl.ds(s * RPT, RPT)])
    pltpu.sync_copy(zb_v, degi_sh.at[pl.ds(s * RPT, RPT)])
    plsc.subcore_barrier()

    def _body(ci, carry):
        pltpu.sync_copy(ones_v, dego_sh.at[src_v.at[ci]], add=True)
        pltpu.sync_copy(ones_v, degi_sh.at[dst_v.at[ci]], add=True)
        return carry

    lax.fori_loop(0, NCHUNK, _body, 0)
    plsc.subcore_barrier()
    pltpu.sync_copy(dego_sh.at[pl.ds(s * RPT, RPT)],
                    dego_hbm.at[c, pl.ds(s * RPT, RPT)])
    pltpu.sync_copy(degi_sh.at[pl.ds(s * RPT, RPT)],
                    degi_hbm.at[c, pl.ds(s * RPT, RPT)])


@functools.partial(
    pl.kernel,
    out_type=jax.ShapeDtypeStruct((NC, N_PAD, D_H), jnp.float32),
    mesh=_MESH,
    scratch_types=[
        pltpu.VMEM((NCHUNK, CH), jnp.int32),
        pltpu.VMEM((NCHUNK, CH), jnp.int32),
        pltpu.VMEM((CH, D_H), jnp.float32),
        pltpu.VMEM((CH, D_H), jnp.float32),
        pltpu.VMEM_SHARED((N_PAD, D_H), jnp.float32),
        pltpu.SemaphoreType.DMA,
        pltpu.SemaphoreType.DMA,
    ],
    compiler_params=_SC_PARAMS,
)
def _sc_gather_scatter(g_hbm, src_hbm, dst_hbm, z_hbm, out_hbm,
                       src_v, dst_v, buf0_v, buf1_v, agg_sh, sem0, sem1):
    c = lax.axis_index("c")
    s = lax.axis_index("s")
    wid = c * NS + s
    pltpu.sync_copy(src_hbm.at[wid], src_v)
    pltpu.sync_copy(dst_hbm.at[wid], dst_v)
    pltpu.sync_copy(z_hbm.at[pl.ds(s * RPT, RPT)],
                    agg_sh.at[pl.ds(s * RPT, RPT)])
    plsc.subcore_barrier()

    bufs = (buf0_v, buf1_v)
    sems = (sem0, sem1)
    pltpu.async_copy(g_hbm.at[src_v.at[0]], bufs[0], sems[0])
    pltpu.async_copy(g_hbm.at[src_v.at[1]], bufs[1], sems[1])

    def _group(gi, carry):
        for b in range(2):
            ci = gi * 2 + b
            pltpu.make_async_copy(g_hbm.at[src_v.at[ci]],
                                  bufs[b], sems[b]).wait()
            pltpu.sync_copy(bufs[b], agg_sh.at[dst_v.at[ci]], add=True)
            nxt = ci + 2

            @pl.when(nxt < NCHUNK)
            def _():
                pltpu.async_copy(g_hbm.at[src_v.at[nxt]], bufs[b], sems[b])
        return carry

    lax.fori_loop(0, NCHUNK // 2, _group, 0)
    plsc.subcore_barrier()
    pltpu.sync_copy(agg_sh.at[pl.ds(s * RPT, RPT)],
                    out_hbm.at[c, pl.ds(s * RPT, RPT)])


def _tc_pre_body(x_ref, w_ref, b_ref, go_ref, gi_ref,
                 h0_ref, g_ref, dsrc_ref, ddst_ref):
    h = jnp.dot(x_ref[...], w_ref[...], preferred_element_type=jnp.float32)
    h = jnp.maximum(h + b_ref[...][None, :], 0.0)
    dego = (go_ref[0, :] + go_ref[1, :])[:N]
    degi = (gi_ref[0, :] + gi_ref[1, :])[:N]
    dsrc = lax.rsqrt(jnp.where(dego > 0, dego, 1.0))[:, None]
    ddst = lax.rsqrt(jnp.where(degi > 0, degi, 1.0))[:, None]
    h0_ref[...] = h
    g_ref[...] = jnp.concatenate(
        [h * dsrc, jnp.zeros((N_PAD - N, D_H), jnp.float32)], axis=0)
    dsrc_ref[...] = dsrc
    ddst_ref[...] = ddst


def _tc_pre(x, fc0_w, fc0_b, dego_p, degi_p):
    return pl.pallas_call(
        _tc_pre_body,
        out_shape=(
            jax.ShapeDtypeStruct((N, D_H), jnp.float32),
            jax.ShapeDtypeStruct((N_PAD, D_H), jnp.float32),
            jax.ShapeDtypeStruct((N, 1), jnp.float32),
            jax.ShapeDtypeStruct((N, 1), jnp.float32),
        ),
    )(x, fc0_w, fc0_b, dego_p, degi_p)


def _tc_layer_body(beta, part_ref, h0_ref, dsrc_ref, ddst_ref, w_ref, g_ref):
    agg = (part_ref[0, :N, :] + part_ref[1, :N, :]) * ddst_ref[...]
    feat = (1.0 - ALPHA) * agg + ALPHA * h0_ref[...]
    t = jnp.dot(feat, w_ref[...], preferred_element_type=jnp.float32)
    h = jnp.maximum((1.0 - beta) * feat + beta * t, 0.0)
    g_ref[...] = jnp.concatenate(
        [h * dsrc_ref[...], jnp.zeros((N_PAD - N, D_H), jnp.float32)], axis=0)


def _tc_layer(part, h0, dsrc, ddst, w, beta):
    return pl.pallas_call(
        functools.partial(_tc_layer_body, beta),
        out_shape=jax.ShapeDtypeStruct((N_PAD, D_H), jnp.float32),
    )(part, h0, dsrc, ddst, w)


def _tc_last_body(beta, part_ref, h0_ref, ddst_ref, w_ref,
                  fc1w_ref, fc1b_ref, out_ref):
    agg = (part_ref[0, :N, :] + part_ref[1, :N, :]) * ddst_ref[...]
    feat = (1.0 - ALPHA) * agg + ALPHA * h0_ref[...]
    t = jnp.dot(feat, w_ref[...], preferred_element_type=jnp.float32)
    h = jnp.maximum((1.0 - beta) * feat + beta * t, 0.0)
    o = jnp.dot(h, fc1w_ref[...], preferred_element_type=jnp.float32)
    out_ref[...] = jnp.maximum(o + fc1b_ref[...][None, :], 0.0)


def _tc_last(part, h0, ddst, w, fc1_w, fc1_b, beta):
    return pl.pallas_call(
        functools.partial(_tc_last_body, beta),
        out_shape=jax.ShapeDtypeStruct((N, N_CLS), jnp.float32),
    )(part, h0, ddst, w, fc1_w, fc1_b)


def kernel(x, edge_index, fc0_w, fc0_b, layer_ws, fc1_w, fc1_b):
    e = edge_index.shape[1]
    pad = ET - e
    pad_idx = N + (jnp.arange(pad, dtype=jnp.int32) % (N_PAD - N))
    src_r = jnp.concatenate([edge_index[0], pad_idx]).reshape(NW, NCHUNK, CH)
    dst_r = jnp.concatenate([edge_index[1], pad_idx]).reshape(NW, NCHUNK, CH)
    zeros2d = jnp.zeros((N_PAD, D_H), jnp.float32)

    dego_p, degi_p = _sc_degrees(src_r, dst_r)
    h0, g, dsrc, ddst = _tc_pre(x, fc0_w, fc0_b, dego_p, degi_p)
    for i in range(NUM_LAYERS - 2):
        beta = float(np.log(LAMBDA / (i + 1) + 1.0))
        part = _sc_gather_scatter(g, src_r, dst_r, zeros2d)
        if i < NUM_LAYERS - 3:
            g = _tc_layer(part, h0, dsrc, ddst, layer_ws[i], beta)
        else:
            out = _tc_last(part, h0, ddst, layer_ws[i], fc1_w, fc1_b, beta)
    return out


# speedup vs baseline: 14.7768x; 1.1109x over previous
import functools

import jax
import jax.numpy as jnp
import numpy as np
from jax import lax
from jax.experimental import pallas as pl
from jax.experimental.pallas import tpu as pltpu
from jax.experimental.pallas import tpu_sc as plsc

N = 10000
D_IN = 128
D_H = 64
N_CLS = 16
NUM_LAYERS = 8
ALPHA = 0.1
LAMBDA = 0.5

NC = 2
NS = 16
NW = NC * NS
CH = 128
NCHUNK = 80
EPT = NCHUNK * CH
ET = NW * EPT
N_PAD = 10240
RPT = N_PAD // NS

_MESH = plsc.VectorSubcoreMesh(core_axis_name="c", subcore_axis_name="s")
_SC_PARAMS = pltpu.CompilerParams(use_tc_tiling_on_sc=False)


@functools.partial(
    pl.kernel,
    out_type=(
        jax.ShapeDtypeStruct((NC, N_PAD), jnp.float32),
        jax.ShapeDtypeStruct((NC, N_PAD), jnp.float32),
    ),
    mesh=_MESH,
    scratch_types=[
        pltpu.VMEM((NCHUNK, CH), jnp.int32),
        pltpu.VMEM((NCHUNK, CH), jnp.int32),
        pltpu.VMEM((CH,), jnp.float32),
        pltpu.VMEM((RPT,), jnp.float32),
        pltpu.VMEM_SHARED((N_PAD,), jnp.float32),
        pltpu.VMEM_SHARED((N_PAD,), jnp.float32),
    ],
    compiler_params=_SC_PARAMS,
)
def _sc_degrees(src_hbm, dst_hbm, dego_hbm, degi_hbm,
                src_v, dst_v, ones_v, zb_v, dego_sh, degi_sh):
    c = lax.axis_index("c")
    s = lax.axis_index("s")
    wid = c * NS + s
    pltpu.sync_copy(src_hbm.at[wid], src_v)
    pltpu.sync_copy(dst_hbm.at[wid], dst_v)
    for j in range(CH // 16):
        ones_v[pl.ds(j * 16, 16)] = jnp.ones((16,), jnp.float32)

    def _zero(i, carry):
        zb_v[pl.ds(i * 16, 16)] = jnp.zeros((16,), jnp.float32)
        return carry

    lax.fori_loop(0, RPT // 16, _zero, 0)
    pltpu.sync_copy(zb_v, dego_sh.at[pl.ds(s * RPT, RPT)])
    pltpu.sync_copy(zb_v, degi_sh.at[pl.ds(s * RPT, RPT)])
    plsc.subcore_barrier()

    def _body(ci, carry):
        pltpu.sync_copy(ones_v, dego_sh.at[src_v.at[ci]], add=True)
        pltpu.sync_copy(ones_v, degi_sh.at[dst_v.at[ci]], add=True)
        return carry

    lax.fori_loop(0, NCHUNK, _body, 0)
    plsc.subcore_barrier()
    pltpu.sync_copy(dego_sh.at[pl.ds(s * RPT, RPT)],
                    dego_hbm.at[c, pl.ds(s * RPT, RPT)])
    pltpu.sync_copy(degi_sh.at[pl.ds(s * RPT, RPT)],
                    degi_hbm.at[c, pl.ds(s * RPT, RPT)])


@functools.partial(
    pl.kernel,
    out_type=jax.ShapeDtypeStruct((NC, N_PAD, D_H), jnp.float32),
    mesh=_MESH,
    scratch_types=[
        pltpu.VMEM((NCHUNK, CH), jnp.int32),
        pltpu.VMEM((NCHUNK, CH), jnp.int32),
        pltpu.VMEM((4, CH, D_H), jnp.float32),
        pltpu.VMEM_SHARED((N_PAD, D_H), jnp.float32),
        [pltpu.SemaphoreType.DMA] * 4,
        [pltpu.SemaphoreType.DMA] * 4,
    ],
    compiler_params=_SC_PARAMS,
)
def _sc_gather_scatter(g_hbm, src_hbm, dst_hbm, z_hbm, out_hbm,
                       src_v, dst_v, buf_v, agg_sh, gsem, ssem):
    c = lax.axis_index("c")
    s = lax.axis_index("s")
    wid = c * NS + s
    pltpu.sync_copy(src_hbm.at[wid], src_v)
    pltpu.sync_copy(dst_hbm.at[wid], dst_v)
    pltpu.sync_copy(z_hbm.at[pl.ds(s * RPT, RPT)],
                    agg_sh.at[pl.ds(s * RPT, RPT)])
    plsc.subcore_barrier()

    pltpu.async_copy(g_hbm.at[src_v.at[0]], buf_v.at[0], gsem[0])
    pltpu.async_copy(g_hbm.at[src_v.at[1]], buf_v.at[1], gsem[1])

    def _group(gi, carry):
        for b in range(4):
            ci = gi * 4 + b

            @pl.when(ci >= 2)
            def _():
                pltpu.make_async_copy(
                    buf_v.at[(b + 2) % 4],
                    agg_sh.at[dst_v.at[ci]],
                    ssem[(b + 2) % 4]).wait()

            @pl.when(ci + 2 < NCHUNK)
            def _():
                pltpu.async_copy(g_hbm.at[src_v.at[ci + 2]],
                                 buf_v.at[(b + 2) % 4], gsem[(b + 2) % 4])

            pltpu.make_async_copy(g_hbm.at[src_v.at[ci]],
                                  buf_v.at[b], gsem[b]).wait()
            pltpu.async_copy(buf_v.at[b], agg_sh.at[dst_v.at[ci]],
                             ssem[b], add=True)
        return carry

    lax.fori_loop(0, NCHUNK // 4, _group, 0)
    for b in (2, 3):
        pltpu.make_async_copy(buf_v.at[b], agg_sh.at[dst_v.at[0]],
                              ssem[b]).wait()
    plsc.subcore_barrier()
    pltpu.sync_copy(agg_sh.at[pl.ds(s * RPT, RPT)],
                    out_hbm.at[c, pl.ds(s * RPT, RPT)])


def _tc_pre_body(x_ref, w_ref, b_ref, go_ref, gi_ref,
                 h0_ref, g_ref, dsrc_ref, ddst_ref):
    h = jnp.dot(x_ref[...], w_ref[...], preferred_element_type=jnp.float32)
    h = jnp.maximum(h + b_ref[...][None, :], 0.0)
    dego = (go_ref[0, :] + go_ref[1, :])[:N]
    degi = (gi_ref[0, :] + gi_ref[1, :])[:N]
    dsrc = lax.rsqrt(jnp.where(dego > 0, dego, 1.0))[:, None]
    ddst = lax.rsqrt(jnp.where(degi > 0, degi, 1.0))[:, None]
    h0_ref[...] = h
    g_ref[...] = jnp.concatenate(
        [h * dsrc, jnp.zeros((N_PAD - N, D_H), jnp.float32)], axis=0)
    dsrc_ref[...] = dsrc
    ddst_ref[...] = ddst


def _tc_pre(x, fc0_w, fc0_b, dego_p, degi_p):
    return pl.pallas_call(
        _tc_pre_body,
        out_shape=(
            jax.ShapeDtypeStruct((N, D_H), jnp.float32),
            jax.ShapeDtypeStruct((N_PAD, D_H), jnp.float32),
            jax.ShapeDtypeStruct((N, 1), jnp.float32),
            jax.ShapeDtypeStruct((N, 1), jnp.float32),
        ),
    )(x, fc0_w, fc0_b, dego_p, degi_p)


def _tc_layer_body(beta, part_ref, h0_ref, dsrc_ref, ddst_ref, w_ref, g_ref):
    agg = (part_ref[0, :N, :] + part_ref[1, :N, :]) * ddst_ref[...]
    feat = (1.0 - ALPHA) * agg + ALPHA * h0_ref[...]
    t = jnp.dot(feat, w_ref[...], preferred_element_type=jnp.float32)
    h = jnp.maximum((1.0 - beta) * feat + beta * t, 0.0)
    g_ref[...] = jnp.concatenate(
        [h * dsrc_ref[...], jnp.zeros((N_PAD - N, D_H), jnp.float32)], axis=0)


def _tc_layer(part, h0, dsrc, ddst, w, beta):
    return pl.pallas_call(
        functools.partial(_tc_layer_body, beta),
        out_shape=jax.ShapeDtypeStruct((N_PAD, D_H), jnp.float32),
    )(part, h0, dsrc, ddst, w)


def _tc_last_body(beta, part_ref, h0_ref, ddst_ref, w_ref,
                  fc1w_ref, fc1b_ref, out_ref):
    agg = (part_ref[0, :N, :] + part_ref[1, :N, :]) * ddst_ref[...]
    feat = (1.0 - ALPHA) * agg + ALPHA * h0_ref[...]
    t = jnp.dot(feat, w_ref[...], preferred_element_type=jnp.float32)
    h = jnp.maximum((1.0 - beta) * feat + beta * t, 0.0)
    o = jnp.dot(h, fc1w_ref[...], preferred_element_type=jnp.float32)
    out_ref[...] = jnp.maximum(o + fc1b_ref[...][None, :], 0.0)


def _tc_last(part, h0, ddst, w, fc1_w, fc1_b, beta):
    return pl.pallas_call(
        functools.partial(_tc_last_body, beta),
        out_shape=jax.ShapeDtypeStruct((N, N_CLS), jnp.float32),
    )(part, h0, ddst, w, fc1_w, fc1_b)


def kernel(x, edge_index, fc0_w, fc0_b, layer_ws, fc1_w, fc1_b):
    e = edge_index.shape[1]
    pad = ET - e
    pad_idx = N + (jnp.arange(pad, dtype=jnp.int32) % (N_PAD - N))
    src_r = jnp.concatenate([edge_index[0], pad_idx]).reshape(NW, NCHUNK, CH)
    dst_r = jnp.concatenate([edge_index[1], pad_idx]).reshape(NW, NCHUNK, CH)
    zeros2d = jnp.zeros((N_PAD, D_H), jnp.float32)

    dego_p, degi_p = _sc_degrees(src_r, dst_r)
    h0, g, dsrc, ddst = _tc_pre(x, fc0_w, fc0_b, dego_p, degi_p)
    for i in range(NUM_LAYERS - 2):
        beta = float(np.log(LAMBDA / (i + 1) + 1.0))
        part = _sc_gather_scatter(g, src_r, dst_r, zeros2d)
        if i < NUM_LAYERS - 3:
            g = _tc_layer(part, h0, dsrc, ddst, layer_ws[i], beta)
        else:
            out = _tc_last(part, h0, ddst, layer_ws[i], fc1_w, fc1_b, beta)
    return out
